# Initial kernel scaffold; baseline (speedup 1.0000x reference)
import jax
import jax.numpy as jnp
from jax.experimental import pallas as pl


def kernel(features, edge_index, edge_weights, W1, b1, W2, b2):
    raise NotImplementedError("write your pallas kernel here")


# speedup vs baseline: 11.8260x; 11.8260x over previous
import functools

import jax
import jax.numpy as jnp
from jax import lax
from jax.experimental import pallas as pl
from jax.experimental.pallas import tpu as pltpu
from jax.experimental.pallas import tpu_sc as plsc

_N = 10000
_NP = 10240
_F = 64
_ALPHA = 0.1
_ITERS = 10

_TILES = 16
_GRP = 128
_SUP = 1024
_HALF = 512
_NCHUNK = 21
_EPT = _SUP * _NCHUNK
_EPAD = _TILES * _EPT
_BROWS = 32


def _mlp_body(f_ref, w1_ref, b1_ref, w2_ref, b2_ref, ws_ref,
              s0_ref, u_ref, v_ref, uf_ref, vf_ref):
    h = jnp.maximum(jnp.dot(f_ref[...], w1_ref[...],
                            preferred_element_type=jnp.float32) + b1_ref[...], 0.0)
    z = jnp.dot(h, w2_ref[...], preferred_element_type=jnp.float32) + b2_ref[...]
    ws = ws_ref[...]
    dinv = jnp.sqrt(ws)
    s0 = dinv * z
    s0_ref[...] = s0
    v_ref[...] = _ALPHA * s0
    u_ref[...] = jnp.broadcast_to((1.0 - _ALPHA) * ws, z.shape)
    uf_ref[...] = jnp.broadcast_to((1.0 - _ALPHA) * dinv, z.shape)
    vf_ref[...] = _ALPHA * z


def _mlp(features, W1, b1, W2, b2, ws):
    bs = 1000
    grid = (_N // bs,)
    outs = [jax.ShapeDtypeStruct((_N, _F), jnp.float32)] * 5
    return pl.pallas_call(
        _mlp_body,
        grid=grid,
        in_specs=[
            pl.BlockSpec((bs, 128), lambda i: (i, 0)),
            pl.BlockSpec((128, _F), lambda i: (0, 0)),
            pl.BlockSpec((1, _F), lambda i: (0, 0)),
            pl.BlockSpec((_F, _F), lambda i: (0, 0)),
            pl.BlockSpec((1, _F), lambda i: (0, 0)),
            pl.BlockSpec((bs, 1), lambda i: (i, 0)),
        ],
        out_specs=[pl.BlockSpec((bs, _F), lambda i: (i, 0))] * 5,
        out_shape=outs,
    )(features, W1, b1, W2, b2, ws)


def _prop(s0, u, v, uf, vf, row2, col2):
    mesh = plsc.VectorSubcoreMesh(core_axis_name="c", subcore_axis_name="s")

    @functools.partial(
        pl.kernel,
        out_type=jax.ShapeDtypeStruct((_NP, _F), jnp.float32),
        mesh=mesh,
        compiler_params=pltpu.CompilerParams(use_tc_tiling_on_sc=False),
        scratch_types=[
            pltpu.VMEM_SHARED((_NP, _F), jnp.float32),
            pltpu.VMEM_SHARED((_NP, _F), jnp.float32),
            pltpu.VMEM((_HALF, _F), jnp.float32),
            pltpu.VMEM((_SUP // _GRP, _GRP), jnp.int32),
            pltpu.VMEM((_SUP // _GRP, _GRP), jnp.int32),
            pltpu.VMEM((_BROWS, _F), jnp.float32),
            pltpu.VMEM((_BROWS, _F), jnp.float32),
            pltpu.VMEM((_BROWS, _F), jnp.float32),
            pltpu.VMEM((_BROWS, _F), jnp.float32),
            pltpu.SemaphoreType.DMA,
            pltpu.SemaphoreType.DMA,
        ],
    )
    def k(s0_h, u_h, v_h, uf_h, vf_h, row_h, col_h, out_h,
          s_sh, acc_sh, gbuf, colbuf, rowbuf, abuf, ubuf, vbuf, zbuf,
          gsem, ssem):
        cid = lax.axis_index("c")
        sid = lax.axis_index("s")

        @pl.when(cid == 0)
        def _work():
            rbase = sid * (_NP // _TILES)
            ebase = sid * (_EPT // _GRP)

            def _zb(i, _):
                zbuf[i, pl.ds(0, 16)] = jnp.zeros((16,), jnp.float32)
                zbuf[i, pl.ds(16, 16)] = jnp.zeros((16,), jnp.float32)
                zbuf[i, pl.ds(32, 16)] = jnp.zeros((16,), jnp.float32)
                zbuf[i, pl.ds(48, 16)] = jnp.zeros((16,), jnp.float32)
                return 0
            lax.fori_loop(0, _BROWS, _zb, 0)

            def _init(b, _):
                rb = rbase + b * _BROWS
                pltpu.sync_copy(zbuf, acc_sh.at[pl.ds(rb, _BROWS)])
                pltpu.sync_copy(s0_h.at[pl.ds(rb, _BROWS)],
                                s_sh.at[pl.ds(rb, _BROWS)])
                return 0
            lax.fori_loop(0, 640 // _BROWS, _init, 0)

            plsc.subcore_barrier()

            def edge_phase():
                def chunk_body(c, _):
                    gb = ebase + c * (_SUP // _GRP)
                    pltpu.sync_copy(col_h.at[pl.ds(gb, _SUP // _GRP)], colbuf)
                    pltpu.sync_copy(row_h.at[pl.ds(gb, _SUP // _GRP)], rowbuf)
                    for h in range(_SUP // _HALF):
                        gws = [
                            pltpu.async_copy(
                                s_sh.at[colbuf.at[h * (_HALF // _GRP) + j]],
                                gbuf.at[pl.ds(j * _GRP, _GRP)], gsem)
                            for j in range(_HALF // _GRP)
                        ]
                        for w in gws:
                            w.wait()
                        sws = [
                            pltpu.async_copy(
                                gbuf.at[pl.ds(j * _GRP, _GRP)],
                                acc_sh.at[rowbuf.at[h * (_HALF // _GRP) + j]],
                                ssem, add=True)
                            for j in range(_HALF // _GRP)
                        ]
                        for w in sws:
                            w.wait()
                    return 0
                lax.fori_loop(0, _NCHUNK, chunk_body, 0)

            def blend_phase(cu_h, cv_h, last):
                def _blc(b, _):
                    rb = rbase + b * _BROWS
                    pltpu.sync_copy(acc_sh.at[pl.ds(rb, _BROWS)], abuf)
                    pltpu.sync_copy(cu_h.at[pl.ds(rb, _BROWS)], ubuf)
                    pltpu.sync_copy(cv_h.at[pl.ds(rb, _BROWS)], vbuf)

                    def _bl(i, _):
                        for q in range(4):
                            sl = pl.ds(q * 16, 16)
                            abuf[i, sl] = abuf[i, sl] * ubuf[i, sl] + vbuf[i, sl]
                        return 0
                    lax.fori_loop(0, _BROWS, _bl, 0)
                    if last:
                        pltpu.sync_copy(abuf, out_h.at[pl.ds(rb, _BROWS)])
                    else:
                        pltpu.sync_copy(abuf, s_sh.at[pl.ds(rb, _BROWS)])
                        pltpu.sync_copy(zbuf, acc_sh.at[pl.ds(rb, _BROWS)])
                    return 0
                lax.fori_loop(0, 640 // _BROWS, _blc, 0)

            def iter_body(it, _):
                edge_phase()
                plsc.subcore_barrier()
                blend_phase(u_h, v_h, last=False)
                plsc.subcore_barrier()
                return 0
            lax.fori_loop(0, _ITERS - 1, iter_body, 0)

            edge_phase()
            plsc.subcore_barrier()
            blend_phase(uf_h, vf_h, last=True)

    return k(s0, u, v, uf, vf, row2, col2)


def _pad_rows(x):
    return jnp.concatenate(
        [x, jnp.zeros((_NP - _N, _F), jnp.float32)], axis=0)


def kernel(features, edge_index, edge_weights, W1, b1, W2, b2):
    row = edge_index[0].astype(jnp.int32)
    col = edge_index[1].astype(jnp.int32)
    et = edge_weights.shape[0]
    ws = edge_weights[et - _N:].reshape(_N, 1)

    pad = _EPAD - et
    ar = jnp.arange(pad, dtype=jnp.int32)
    prow = _N + (ar % (_NP - _N))
    pcol = ar % _N
    row2 = jnp.concatenate([row, prow]).reshape(_EPAD // _GRP, _GRP)
    col2 = jnp.concatenate([col, pcol]).reshape(_EPAD // _GRP, _GRP)

    s0, u, v, uf, vf = _mlp(features, W1, b1.reshape(1, _F), W2,
                            b2.reshape(1, _F), ws)
    out = _prop(_pad_rows(s0), _pad_rows(u), _pad_rows(v),
                _pad_rows(uf), _pad_rows(vf), row2, col2)
    return out[:_N]


# speedup vs baseline: 18.4158x; 1.5572x over previous
import functools

import jax
import jax.numpy as jnp
from jax import lax
from jax.experimental import pallas as pl
from jax.experimental.pallas import tpu as pltpu
from jax.experimental.pallas import tpu_sc as plsc

_N = 10000
_NP = 10240
_F = 64
_ALPHA = 0.1
_ITERS = 10

_TILES = 16
_GRP = 128
_G4 = 4
_HALF = _G4 * _GRP
_SUP = 2 * _HALF
_GPC = _SUP // _GRP
_NCHUNK = 21
_EPT = _SUP * _NCHUNK
_EPAD = _TILES * _EPT
_BROWS = 64


def _mlp_body(f_ref, w1_ref, b1_ref, w2_ref, b2_ref, ws_ref,
              s0_ref, u_ref, v_ref, uf_ref, vf_ref):
    h = jnp.maximum(jnp.dot(f_ref[...], w1_ref[...],
                            preferred_element_type=jnp.float32) + b1_ref[...], 0.0)
    z = jnp.dot(h, w2_ref[...], preferred_element_type=jnp.float32) + b2_ref[...]
    ws = ws_ref[...]
    dinv = jnp.sqrt(ws)
    s0 = dinv * z
    s0_ref[...] = s0
    v_ref[...] = _ALPHA * s0
    u_ref[...] = jnp.broadcast_to((1.0 - _ALPHA) * ws, z.shape)
    uf_ref[...] = jnp.broadcast_to((1.0 - _ALPHA) * dinv, z.shape)
    vf_ref[...] = _ALPHA * z


def _mlp(features, W1, b1, W2, b2, ws):
    bs = 1000
    grid = (_N // bs,)
    outs = [jax.ShapeDtypeStruct((_N, _F), jnp.float32)] * 5
    return pl.pallas_call(
        _mlp_body,
        grid=grid,
        in_specs=[
            pl.BlockSpec((bs, 128), lambda i: (i, 0)),
            pl.BlockSpec((128, _F), lambda i: (0, 0)),
            pl.BlockSpec((1, _F), lambda i: (0, 0)),
            pl.BlockSpec((_F, _F), lambda i: (0, 0)),
            pl.BlockSpec((1, _F), lambda i: (0, 0)),
            pl.BlockSpec((bs, 1), lambda i: (i, 0)),
        ],
        out_specs=[pl.BlockSpec((bs, _F), lambda i: (i, 0))] * 5,
        out_shape=outs,
    )(features, W1, b1, W2, b2, ws)


def _prop(s0, u, v, uf, vf, row2, col2):
    mesh = plsc.VectorSubcoreMesh(core_axis_name="c", subcore_axis_name="s")

    @functools.partial(
        pl.kernel,
        out_type=(
            jax.ShapeDtypeStruct((_NP, _F), jnp.float32),
            jax.ShapeDtypeStruct((_NP, _F), jnp.float32),
            jax.ShapeDtypeStruct((_NP, _F), jnp.float32),
        ),
        mesh=mesh,
        compiler_params=pltpu.CompilerParams(use_tc_tiling_on_sc=False),
        scratch_types=[
            pltpu.VMEM_SHARED((_NP, _F), jnp.float32),
            pltpu.VMEM((_HALF, _F), jnp.float32),
            pltpu.VMEM((_HALF, _F), jnp.float32),
            pltpu.VMEM((2 * _GPC, _GRP), jnp.int32),
            pltpu.VMEM((2 * _GPC, _GRP), jnp.int32),
            pltpu.VMEM((_BROWS, _F), jnp.float32),
            pltpu.VMEM((_BROWS, _F), jnp.float32),
            pltpu.VMEM((_BROWS, _F), jnp.float32),
            pltpu.VMEM((_BROWS, _F), jnp.float32),
            pltpu.SemaphoreType.DMA,
            pltpu.SemaphoreType.DMA,
            pltpu.SemaphoreType.DMA,
            pltpu.SemaphoreType.DMA,
            pltpu.SemaphoreType.DMA,
        ],
    )
    def k(s0_h, u_h, v_h, uf_h, vf_h, row_h, col_h, sa_h, sb_h, out_h,
          acc_sh, gbuf0, gbuf1, colbuf, rowbuf, abuf, ubuf, vbuf, zbuf,
          gsem0, gsem1, ssem0, ssem1, isem):
        cid = lax.axis_index("c")
        sid = lax.axis_index("s")

        @pl.when(cid == 0)
        def _work():
            rbase = sid * (_NP // _TILES)
            ebase = sid * (_EPT // _GRP)

            def drain(buf, sem):
                pltpu.make_async_copy(sa_h.at[pl.ds(0, _HALF)], buf, sem).wait()

            def _zb(i, _):
                for q in range(4):
                    zbuf[i, pl.ds(q * 16, 16)] = jnp.zeros((16,), jnp.float32)
                return 0
            lax.fori_loop(0, _BROWS, _zb, 0)

            def _init(b, _):
                pltpu.sync_copy(zbuf, acc_sh.at[pl.ds(rbase + b * _BROWS, _BROWS)])
                return 0
            lax.fori_loop(0, 640 // _BROWS, _init, 0)

            plsc.subcore_barrier()

            def edge_phase(src_h):
                pltpu.sync_copy(col_h.at[pl.ds(ebase, _GPC)],
                                colbuf.at[pl.ds(0, _GPC)])
                pltpu.sync_copy(row_h.at[pl.ds(ebase, _GPC)],
                                rowbuf.at[pl.ds(0, _GPC)])

                def chunk(c, _):
                    slot = lax.rem(c, 2)
                    nslot = lax.rem(c + 1, 2)

                    @pl.when(c > 0)
                    def _wi():
                        pltpu.make_async_copy(
                            col_h.at[pl.ds(ebase, _GPC)],
                            colbuf.at[pl.ds(0, _GPC)], isem).wait()
                        pltpu.make_async_copy(
                            row_h.at[pl.ds(ebase, _GPC)],
                            rowbuf.at[pl.ds(0, _GPC)], isem).wait()

                    @pl.when(c > 0)
                    def _ds0():
                        drain(gbuf0, ssem0)
                    for j in range(_G4):
                        pltpu.async_copy(src_h.at[colbuf.at[slot * _GPC + j]],
                                         gbuf0.at[pl.ds(j * _GRP, _GRP)], gsem0)

                    @pl.when(c > 0)
                    def _sg1():
                        drain(gbuf1, gsem1)
                        pslot = lax.rem(c + 1, 2)
                        for j in range(_G4):
                            pltpu.async_copy(
                                gbuf1.at[pl.ds(j * _GRP, _GRP)],
                                acc_sh.at[rowbuf.at[pslot * _GPC + _G4 + j]],
                                ssem1, add=True)
                        drain(gbuf1, ssem1)

                    @pl.when(c < _NCHUNK - 1)
                    def _pf():
                        gb = ebase + (c + 1) * _GPC
                        pltpu.async_copy(col_h.at[pl.ds(gb, _GPC)],
                                         colbuf.at[pl.ds(nslot * _GPC, _GPC)],
                                         isem)
                        pltpu.async_copy(row_h.at[pl.ds(gb, _GPC)],
                                         rowbuf.at[pl.ds(nslot * _GPC, _GPC)],
                                         isem)

                    for j in range(_G4):
                        pltpu.async_copy(
                            src_h.at[colbuf.at[slot * _GPC + _G4 + j]],
                            gbuf1.at[pl.ds(j * _GRP, _GRP)], gsem1)

                    drain(gbuf0, gsem0)
                    for j in range(_G4):
                        pltpu.async_copy(
                            gbuf0.at[pl.ds(j * _GRP, _GRP)],
                            acc_sh.at[rowbuf.at[slot * _GPC + j]],
                            ssem0, add=True)
                    return 0
                lax.fori_loop(0, _NCHUNK, chunk, 0)

                drain(gbuf1, gsem1)
                lslot = (_NCHUNK - 1) % 2
                for j in range(_G4):
                    pltpu.async_copy(
                        gbuf1.at[pl.ds(j * _GRP, _GRP)],
                        acc_sh.at[rowbuf.at[lslot * _GPC + _G4 + j]],
                        ssem1, add=True)
                drain(gbuf0, ssem0)
                drain(gbuf1, ssem1)

            def blend_phase(cu_h, cv_h, dst_h, last):
                def _blc(b, _):
                    rb = rbase + b * _BROWS
                    pltpu.sync_copy(acc_sh.at[pl.ds(rb, _BROWS)], abuf)
                    pltpu.sync_copy(cu_h.at[pl.ds(rb, _BROWS)], ubuf)
                    pltpu.sync_copy(cv_h.at[pl.ds(rb, _BROWS)], vbuf)

                    def _bl(i, _):
                        for q in range(4):
                            sl = pl.ds(q * 16, 16)
                            abuf[i, sl] = abuf[i, sl] * ubuf[i, sl] + vbuf[i, sl]
                        return 0
                    lax.fori_loop(0, _BROWS, _bl, 0)
                    pltpu.sync_copy(abuf, dst_h.at[pl.ds(rb, _BROWS)])
                    if not last:
                        pltpu.sync_copy(zbuf, acc_sh.at[pl.ds(rb, _BROWS)])
                    return 0
                lax.fori_loop(0, 640 // _BROWS, _blc, 0)

            def iter_body(kk, _):
                @pl.when(kk == 0)
                def _e0():
                    edge_phase(s0_h)

                @pl.when(lax.rem(kk, 2) == 1)
                def _eo():
                    edge_phase(sa_h)

                @pl.when((kk > 0) & (lax.rem(kk, 2) == 0))
                def _ee():
                    edge_phase(sb_h)

                plsc.subcore_barrier()

                @pl.when(lax.rem(kk, 2) == 0)
                def _b0():
                    blend_phase(u_h, v_h, sa_h, last=False)

                @pl.when(lax.rem(kk, 2) == 1)
                def _b1():
                    blend_phase(u_h, v_h, sb_h, last=False)

                plsc.subcore_barrier()
                return 0
            lax.fori_loop(0, _ITERS - 1, iter_body, 0)

            edge_phase(sa_h)
            plsc.subcore_barrier()
            blend_phase(uf_h, vf_h, out_h, last=True)

    return k(s0, u, v, uf, vf, row2, col2)


def _pad_rows(x):
    return jnp.concatenate(
        [x, jnp.zeros((_NP - _N, _F), jnp.float32)], axis=0)


def kernel(features, edge_index, edge_weights, W1, b1, W2, b2):
    row = edge_index[0].astype(jnp.int32)
    col = edge_index[1].astype(jnp.int32)
    et = edge_weights.shape[0]
    ws = edge_weights[et - _N:].reshape(_N, 1)

    pad = _EPAD - et
    ar = jnp.arange(pad, dtype=jnp.int32)
    prow = _N + (ar % (_NP - _N))
    pcol = ar % _N
    row2 = jnp.concatenate([row, prow]).reshape(_EPAD // _GRP, _GRP)
    col2 = jnp.concatenate([col, pcol]).reshape(_EPAD // _GRP, _GRP)

    s0, u, v, uf, vf = _mlp(features, W1, b1.reshape(1, _F), W2,
                            b2.reshape(1, _F), ws)
    _, _, out = _prop(_pad_rows(s0), _pad_rows(u), _pad_rows(v),
                      _pad_rows(uf), _pad_rows(vf), row2, col2)
    return out[:_N]


# speedup vs baseline: 19.5616x; 1.0622x over previous
import functools

import jax
import jax.numpy as jnp
from jax import lax
from jax.experimental import pallas as pl
from jax.experimental.pallas import tpu as pltpu
from jax.experimental.pallas import tpu_sc as plsc

_N = 10000
_NP = 10240
_HN = _NP // 2
_ACC_ROWS = _HN + 128
_F = 64
_ALPHA = 0.1
_ITERS = 10

_TILES = 16
_GRP = 128
_G4 = 4
_HALF = _G4 * _GRP
_SUP = 2 * _HALF
_GPC = _SUP // _GRP
_NCHUNK = 21
_EPT = _SUP * _NCHUNK
_EPAD = _TILES * _EPT
_BROWS = 64


def _mlp_body(f_ref, w1_ref, b1_ref, w2_ref, b2_ref, ws_ref,
              s0_ref, u_ref, v_ref, uf_ref, vf_ref):
    h = jnp.maximum(jnp.dot(f_ref[...], w1_ref[...],
                            preferred_element_type=jnp.float32) + b1_ref[...], 0.0)
    z = jnp.dot(h, w2_ref[...], preferred_element_type=jnp.float32) + b2_ref[...]
    ws = ws_ref[...]
    dinv = jnp.sqrt(ws)
    s0 = dinv * z
    s0_ref[...] = s0
    v_ref[...] = _ALPHA * s0
    u_ref[...] = jnp.broadcast_to((1.0 - _ALPHA) * ws, z.shape)
    uf_ref[...] = jnp.broadcast_to((1.0 - _ALPHA) * dinv, z.shape)
    vf_ref[...] = _ALPHA * z


def _mlp(features, W1, b1, W2, b2, ws):
    bs = 1000
    grid = (_N // bs,)
    outs = [jax.ShapeDtypeStruct((_N, _F), jnp.float32)] * 5
    return pl.pallas_call(
        _mlp_body,
        grid=grid,
        in_specs=[
            pl.BlockSpec((bs, 128), lambda i: (i, 0)),
            pl.BlockSpec((128, _F), lambda i: (0, 0)),
            pl.BlockSpec((1, _F), lambda i: (0, 0)),
            pl.BlockSpec((_F, _F), lambda i: (0, 0)),
            pl.BlockSpec((1, _F), lambda i: (0, 0)),
            pl.BlockSpec((bs, 1), lambda i: (i, 0)),
        ],
        out_specs=[pl.BlockSpec((bs, _F), lambda i: (i, 0))] * 5,
        out_shape=outs,
    )(features, W1, b1, W2, b2, ws)


_MESH = plsc.VectorSubcoreMesh(core_axis_name="c", subcore_axis_name="s")


@functools.partial(
    pl.kernel,
    out_type=jax.ShapeDtypeStruct((_NP, _F), jnp.float32),
    mesh=_MESH,
    compiler_params=pltpu.CompilerParams(use_tc_tiling_on_sc=False),
    scratch_types=[
        pltpu.VMEM_SHARED((_ACC_ROWS, _F), jnp.float32),
        pltpu.VMEM((_HALF, _F), jnp.float32),
        pltpu.VMEM((_HALF, _F), jnp.float32),
        pltpu.VMEM((2 * _GPC, _GRP), jnp.int32),
        pltpu.VMEM((2 * _GPC, _GRP), jnp.int32),
        pltpu.VMEM((_BROWS, _F), jnp.float32),
        pltpu.VMEM((_BROWS, _F), jnp.float32),
        pltpu.VMEM((_BROWS, _F), jnp.float32),
        pltpu.VMEM((_BROWS, _F), jnp.float32),
        pltpu.SemaphoreType.DMA,
        pltpu.SemaphoreType.DMA,
        pltpu.SemaphoreType.DMA,
        pltpu.SemaphoreType.DMA,
        pltpu.SemaphoreType.DMA,
    ],
)
def _prop_iter(src_h, u_h, v_h, rowc0_h, rowc1_h, col_h, dst_h,
               acc_sh, gbuf0, gbuf1, colbuf, rowbuf, abuf, ubuf, vbuf, zbuf,
               gsem0, gsem1, ssem0, ssem1, isem):
    cid = lax.axis_index("c")
    sid = lax.axis_index("s")

    def work(row_h):
        ebase = sid * (_EPT // _GRP)
        lbase = sid * (_HN // _TILES)
        hbase = cid * _HN + lbase

        def drain(buf, sem):
            pltpu.make_async_copy(src_h.at[pl.ds(0, _HALF)], buf, sem).wait()

        def _zb(i, _):
            for q in range(4):
                zbuf[i, pl.ds(q * 16, 16)] = jnp.zeros((16,), jnp.float32)
            return 0
        lax.fori_loop(0, _BROWS, _zb, 0)

        zb = sid * (_ACC_ROWS // _TILES)
        def _init(b, _):
            pltpu.sync_copy(zbuf, acc_sh.at[pl.ds(zb + b * _BROWS, _BROWS)])
            return 0
        lax.fori_loop(0, 5, _init, 0)
        pltpu.sync_copy(zbuf.at[pl.ds(0, 8)],
                        acc_sh.at[pl.ds(zb + 5 * _BROWS, 8)])

        plsc.subcore_barrier()

        pltpu.sync_copy(col_h.at[pl.ds(ebase, _GPC)], colbuf.at[pl.ds(0, _GPC)])
        pltpu.sync_copy(row_h.at[pl.ds(ebase, _GPC)], rowbuf.at[pl.ds(0, _GPC)])

        def chunk(c, _):
            slot = lax.rem(c, 2)
            nslot = lax.rem(c + 1, 2)

            @pl.when(c > 0)
            def _wi():
                pltpu.make_async_copy(col_h.at[pl.ds(ebase, _GPC)],
                                      colbuf.at[pl.ds(0, _GPC)], isem).wait()
                pltpu.make_async_copy(row_h.at[pl.ds(ebase, _GPC)],
                                      rowbuf.at[pl.ds(0, _GPC)], isem).wait()

            @pl.when(c > 0)
            def _ds0():
                drain(gbuf0, ssem0)
            for j in range(_G4):
                pltpu.async_copy(src_h.at[colbuf.at[slot * _GPC + j]],
                                 gbuf0.at[pl.ds(j * _GRP, _GRP)], gsem0)

            @pl.when(c > 0)
            def _sg1():
                drain(gbuf1, gsem1)
                pslot = lax.rem(c + 1, 2)
                for j in range(_G4):
                    pltpu.async_copy(
                        gbuf1.at[pl.ds(j * _GRP, _GRP)],
                        acc_sh.at[rowbuf.at[pslot * _GPC + _G4 + j]],
                        ssem1, add=True)
                drain(gbuf1, ssem1)

            @pl.when(c < _NCHUNK - 1)
            def _pf():
                gb = ebase + (c + 1) * _GPC
                pltpu.async_copy(col_h.at[pl.ds(gb, _GPC)],
                                 colbuf.at[pl.ds(nslot * _GPC, _GPC)], isem)
                pltpu.async_copy(row_h.at[pl.ds(gb, _GPC)],
                                 rowbuf.at[pl.ds(nslot * _GPC, _GPC)], isem)

            for j in range(_G4):
                pltpu.async_copy(src_h.at[colbuf.at[slot * _GPC + _G4 + j]],
                                 gbuf1.at[pl.ds(j * _GRP, _GRP)], gsem1)

            drain(gbuf0, gsem0)
            for j in range(_G4):
                pltpu.async_copy(gbuf0.at[pl.ds(j * _GRP, _GRP)],
                                 acc_sh.at[rowbuf.at[slot * _GPC + j]],
                                 ssem0, add=True)
            return 0
        lax.fori_loop(0, _NCHUNK, chunk, 0)

        drain(gbuf1, gsem1)
        lslot = (_NCHUNK - 1) % 2
        for j in range(_G4):
            pltpu.async_copy(gbuf1.at[pl.ds(j * _GRP, _GRP)],
                             acc_sh.at[rowbuf.at[lslot * _GPC + _G4 + j]],
                             ssem1, add=True)
        drain(gbuf0, ssem0)
        drain(gbuf1, ssem1)

        plsc.subcore_barrier()

        def _blc(b, _):
            lb = lbase + b * _BROWS
            hb = hbase + b * _BROWS
            pltpu.sync_copy(acc_sh.at[pl.ds(lb, _BROWS)], abuf)
            pltpu.sync_copy(u_h.at[pl.ds(hb, _BROWS)], ubuf)
            pltpu.sync_copy(v_h.at[pl.ds(hb, _BROWS)], vbuf)

            def _bl(i, _):
                for q in range(4):
                    sl = pl.ds(q * 16, 16)
                    abuf[i, sl] = abuf[i, sl] * ubuf[i, sl] + vbuf[i, sl]
                return 0
            lax.fori_loop(0, _BROWS, _bl, 0)
            pltpu.sync_copy(abuf, dst_h.at[pl.ds(hb, _BROWS)])
            return 0
        lax.fori_loop(0, _HN // _TILES // _BROWS, _blc, 0)

    @pl.when(cid == 0)
    def _c0():
        work(rowc0_h)

    @pl.when(cid == 1)
    def _c1():
        work(rowc1_h)


def _pad_rows(x):
    return jnp.concatenate(
        [x, jnp.zeros((_NP - _N, _F), jnp.float32)], axis=0)


def _remap(row, core):
    t = row - core * _HN
    own = (t >= 0) & (t < _HN)
    return jnp.where(own, t, _HN + (row & 127)).astype(jnp.int32)


def kernel(features, edge_index, edge_weights, W1, b1, W2, b2):
    row = edge_index[0].astype(jnp.int32)
    col = edge_index[1].astype(jnp.int32)
    et = edge_weights.shape[0]
    ws = edge_weights[et - _N:].reshape(_N, 1)

    pad = _EPAD - et
    ar = jnp.arange(pad, dtype=jnp.int32)
    prow = _NP + (ar % 128)
    pcol = ar % _N
    rowp = jnp.concatenate([row, prow])
    rowc0 = _remap(rowp, 0).reshape(_EPAD // _GRP, _GRP)
    rowc1 = _remap(rowp, 1).reshape(_EPAD // _GRP, _GRP)
    col2 = jnp.concatenate([col, pcol]).reshape(_EPAD // _GRP, _GRP)

    s0, u, v, uf, vf = _mlp(features, W1, b1.reshape(1, _F), W2,
                            b2.reshape(1, _F), ws)
    s0, u, v, uf, vf = (_pad_rows(a) for a in (s0, u, v, uf, vf))
    s = s0
    for _ in range(_ITERS - 1):
        s = _prop_iter(s, u, v, rowc0, rowc1, col2)
    out = _prop_iter(s, uf, vf, rowc0, rowc1, col2)
    return out[:_N]
